# baseline (speedup 1.0000x reference)
import functools

import jax
import jax.numpy as jnp
import numpy as np
from jax import lax
from jax.experimental import pallas as pl
from jax.experimental.pallas import tpu as pltpu
from jax.experimental.pallas import tpu_sc as plsc

V = 100000
F = 26
D = 32
DENSE_IN = 13
B = 4096
NF = F + 1
NPAIR = NF * (NF - 1) // 2

BF = B * F

try:
    _info = plsc.get_sparse_core_info()
    _NC, _NS = _info.num_cores, _info.num_subcores
except ValueError:
    _NC, _NS = 2, 16
NW = _NC * _NS
BPW = BF // NW
CH = 128
NCH = BPW // CH


def _sc_gather_body(table_hbm, idxT_hbm, out_hbm, idx_v, rows_v, sem):
    wid = lax.axis_index("s") * _NC + lax.axis_index("c")
    pltpu.sync_copy(idxT_hbm.at[:, pl.ds(wid * CH, CH)], idx_v)
    for lo in range(0, F, 13):
        hi = min(lo + 13, F)
        copies = [
            pltpu.async_copy(
                table_hbm.at[pl.ds(j * V, V)].at[idx_v.at[j]],
                rows_v.at[pl.ds(j * CH, CH), :],
                sem,
            )
            for j in range(lo, hi)
        ]
        for c in copies:
            c.wait()
    for j in range(F):
        pltpu.sync_copy(
            rows_v.at[pl.ds(j * CH, CH), :],
            out_hbm.at[pl.ds(wid * CH, CH), j, :],
        )


@functools.cache
def _sc_gather():
    return pl.kernel(
        _sc_gather_body,
        mesh=plsc.VectorSubcoreMesh(core_axis_name="c", subcore_axis_name="s"),
        out_type=jax.ShapeDtypeStruct((B, F, D), jnp.float32),
        scratch_types=[
            pltpu.VMEM((F, CH), jnp.int32),
            pltpu.VMEM((BPW, D), jnp.float32),
            pltpu.SemaphoreType.DMA,
        ],
        compiler_params=pltpu.CompilerParams(use_tc_tiling_on_sc=False),
    )


BB = 512
NBLK = B // BB


def _tc_body(dense_ref, sparse_ref, dW0, db0, dW1, db1, dW2, db2,
             oW0a, Wt, ob0, oW1, ob1, oW2, ob2, oW3, ob3, out_ref):
    f32 = jnp.float32
    h = dense_ref[:]
    h = jnp.maximum(jnp.dot(h, dW0[:], preferred_element_type=f32) + db0[:], 0.0)
    h = jnp.maximum(jnp.dot(h, dW1[:], preferred_element_type=f32) + db1[:], 0.0)
    h = jnp.maximum(jnp.dot(h, dW2[:], preferred_element_type=f32) + db2[:], 0.0)
    X = jnp.concatenate([h, sparse_ref[:]], axis=1)
    r1 = lax.broadcasted_iota(jnp.int32, (D, NF * D), 1)
    r0 = lax.broadcasted_iota(jnp.int32, (D, NF * D), 0)
    E = (r1 % D == r0).astype(f32)
    s0 = lax.broadcasted_iota(jnp.int32, (NF * D, NF), 0)
    s1 = lax.broadcasted_iota(jnp.int32, (NF * D, NF), 1)
    ONES = (s0 // D == s1).astype(f32)
    acc = jnp.dot(h, oW0a[:], preferred_element_type=f32) + ob0[:]
    for n in range(NF):
        Xn = X[:, D * n:D * (n + 1)]
        Tn = jnp.dot(Xn, E, preferred_element_type=f32)
        Pn = X * Tn
        Gn = jnp.dot(Pn, ONES, preferred_element_type=f32)
        acc = acc + jnp.dot(Gn, Wt[n], preferred_element_type=f32)
    h = jnp.maximum(acc, 0.0)
    h = jnp.maximum(jnp.dot(h, oW1[:], preferred_element_type=f32) + ob1[:], 0.0)
    h = jnp.maximum(jnp.dot(h, oW2[:], preferred_element_type=f32) + ob2[:], 0.0)
    out_ref[:] = jnp.dot(h, oW3[:], preferred_element_type=f32) + ob3[:]


def _full(shape):
    nd = len(shape)
    return pl.BlockSpec(shape, lambda i, _nd=nd: (0,) * _nd)


def kernel(dense_features, sparse_indices, table, dW0, db0, dW1, db1, dW2, db2,
           oW0, ob0, oW1, ob1, oW2, ob2, oW3, ob3):
    siT = sparse_indices.astype(jnp.int32).T
    gathered = _sc_gather()(table, siT)
    sparse_flat = gathered.reshape(B, F * D)

    ti0, ti1 = np.triu_indices(NF, 1)
    P = np.zeros((NF * NF, NPAIR), dtype=np.float32)
    P[ti0 * NF + ti1, np.arange(NPAIR)] = 1.0
    Wt = (jnp.asarray(P) @ oW0[D:]).reshape(NF, NF, oW0.shape[1])

    b2 = lambda x: x.reshape(1, -1)
    grid_spec = pl.GridSpec(
        grid=(NBLK,),
        in_specs=[
            pl.BlockSpec((BB, DENSE_IN), lambda i: (i, 0)),
            pl.BlockSpec((BB, F * D), lambda i: (i, 0)),
            _full(dW0.shape), _full((1, 512)),
            _full(dW1.shape), _full((1, 256)),
            _full(dW2.shape), _full((1, D)),
            _full((D, 512)), _full(Wt.shape), _full((1, 512)),
            _full(oW1.shape), _full((1, 512)),
            _full(oW2.shape), _full((1, 256)),
            _full(oW3.shape), _full((1, 1)),
        ],
        out_specs=pl.BlockSpec((BB, 1), lambda i: (i, 0)),
    )
    logits = pl.pallas_call(
        _tc_body,
        grid_spec=grid_spec,
        out_shape=jax.ShapeDtypeStruct((B, 1), jnp.float32),
    )(dense_features, sparse_flat, dW0, b2(db0), dW1, b2(db1), dW2, b2(db2),
      oW0[:D], Wt, b2(ob0), oW1, b2(ob1), oW2, b2(ob2), oW3, b2(ob3))
    return logits


# speedup vs baseline: 1.0001x; 1.0001x over previous
import functools

import jax
import jax.numpy as jnp
import numpy as np
from jax import lax
from jax.experimental import pallas as pl
from jax.experimental.pallas import tpu as pltpu
from jax.experimental.pallas import tpu_sc as plsc

V = 100000
F = 26
D = 32
DENSE_IN = 13
B = 4096
NF = F + 1
NPAIR = NF * (NF - 1) // 2

BF = B * F

try:
    _info = plsc.get_sparse_core_info()
    _NC, _NS = _info.num_cores, _info.num_subcores
except ValueError:
    _NC, _NS = 2, 16
NW = _NC * _NS
BPW = BF // NW
CH = 128
NCH = BPW // CH


def _sc_gather_body(table_hbm, idxT_hbm, out_hbm, idx_v, rows_v, sem):
    wid = lax.axis_index("s") * _NC + lax.axis_index("c")
    pltpu.sync_copy(idxT_hbm.at[:, wid, :, :], idx_v)
    for lo in range(0, F, 13):
        hi = min(lo + 13, F)
        copies = [
            pltpu.async_copy(
                table_hbm.at[pl.ds(j * V, V)].at[idx_v.at[j // 8, j % 8]],
                rows_v.at[pl.ds(j * CH, CH), :],
                sem,
            )
            for j in range(lo, hi)
        ]
        for c in copies:
            c.wait()
    for j in range(F):
        pltpu.sync_copy(
            rows_v.at[pl.ds(j * CH, CH), :],
            out_hbm.at[pl.ds(wid * CH, CH), j, :],
        )


@functools.cache
def _sc_gather():
    return pl.kernel(
        _sc_gather_body,
        mesh=plsc.VectorSubcoreMesh(core_axis_name="c", subcore_axis_name="s"),
        out_type=jax.ShapeDtypeStruct((B, F, D), jnp.float32),
        scratch_types=[
            pltpu.VMEM((4, 8, CH), jnp.int32),
            pltpu.VMEM((BPW, D), jnp.float32),
            pltpu.SemaphoreType.DMA,
        ],
        compiler_params=pltpu.CompilerParams(use_tc_tiling_on_sc=False),
    )


BB = 512
NBLK = B // BB


def _tc_body(dense_ref, sparse_ref, dW0, db0, dW1, db1, dW2, db2,
             oW0a, Wt, ob0, oW1, ob1, oW2, ob2, oW3, ob3, out_ref):
    f32 = jnp.float32
    h = dense_ref[:]
    h = jnp.maximum(jnp.dot(h, dW0[:], preferred_element_type=f32) + db0[:], 0.0)
    h = jnp.maximum(jnp.dot(h, dW1[:], preferred_element_type=f32) + db1[:], 0.0)
    h = jnp.maximum(jnp.dot(h, dW2[:], preferred_element_type=f32) + db2[:], 0.0)
    X = jnp.concatenate([h, sparse_ref[:]], axis=1)
    r1 = lax.broadcasted_iota(jnp.int32, (D, NF * D), 1)
    r0 = lax.broadcasted_iota(jnp.int32, (D, NF * D), 0)
    E = (r1 % D == r0).astype(f32)
    s0 = lax.broadcasted_iota(jnp.int32, (NF * D, NF), 0)
    s1 = lax.broadcasted_iota(jnp.int32, (NF * D, NF), 1)
    ONES = (s0 // D == s1).astype(f32)
    acc = jnp.dot(h, oW0a[:], preferred_element_type=f32) + ob0[:]
    for n in range(NF):
        Xn = X[:, D * n:D * (n + 1)]
        Tn = jnp.dot(Xn, E, preferred_element_type=f32)
        Pn = X * Tn
        Gn = jnp.dot(Pn, ONES, preferred_element_type=f32)
        acc = acc + jnp.dot(Gn, Wt[n], preferred_element_type=f32)
    h = jnp.maximum(acc, 0.0)
    h = jnp.maximum(jnp.dot(h, oW1[:], preferred_element_type=f32) + ob1[:], 0.0)
    h = jnp.maximum(jnp.dot(h, oW2[:], preferred_element_type=f32) + ob2[:], 0.0)
    out_ref[:] = jnp.dot(h, oW3[:], preferred_element_type=f32) + ob3[:]


def _full(shape):
    nd = len(shape)
    return pl.BlockSpec(shape, lambda i, _nd=nd: (0,) * _nd)


def kernel(dense_features, sparse_indices, table, dW0, db0, dW1, db1, dW2, db2,
           oW0, ob0, oW1, ob1, oW2, ob2, oW3, ob3):
    si = sparse_indices.astype(jnp.int32)
    sip = jnp.pad(si, ((0, 0), (0, 32 - F)))
    idx4 = sip.reshape(32, 128, 4, 8).transpose(2, 0, 3, 1)
    gathered = _sc_gather()(table, idx4)
    sparse_flat = gathered.reshape(B, F * D)

    ti0, ti1 = np.triu_indices(NF, 1)
    P = np.zeros((NF * NF, NPAIR), dtype=np.float32)
    P[ti0 * NF + ti1, np.arange(NPAIR)] = 1.0
    Wt = (jnp.asarray(P) @ oW0[D:]).reshape(NF, NF, oW0.shape[1])

    b2 = lambda x: x.reshape(1, -1)
    grid_spec = pl.GridSpec(
        grid=(NBLK,),
        in_specs=[
            pl.BlockSpec((BB, DENSE_IN), lambda i: (i, 0)),
            pl.BlockSpec((BB, F * D), lambda i: (i, 0)),
            _full(dW0.shape), _full((1, 512)),
            _full(dW1.shape), _full((1, 256)),
            _full(dW2.shape), _full((1, D)),
            _full((D, 512)), _full(Wt.shape), _full((1, 512)),
            _full(oW1.shape), _full((1, 512)),
            _full(oW2.shape), _full((1, 256)),
            _full(oW3.shape), _full((1, 1)),
        ],
        out_specs=pl.BlockSpec((BB, 1), lambda i: (i, 0)),
    )
    logits = pl.pallas_call(
        _tc_body,
        grid_spec=grid_spec,
        out_shape=jax.ShapeDtypeStruct((B, 1), jnp.float32),
    )(dense_features, sparse_flat, dW0, b2(db0), dW1, b2(db1), dW2, b2(db2),
      oW0[:D], Wt, b2(ob0), oW1, b2(ob1), oW2, b2(ob2), oW3, b2(ob3))
    return logits


# speedup vs baseline: 1.5085x; 1.5084x over previous
import functools

import jax
import jax.numpy as jnp
import numpy as np
from jax import lax
from jax.experimental import pallas as pl
from jax.experimental.pallas import tpu as pltpu
from jax.experimental.pallas import tpu_sc as plsc

V = 100000
F = 26
D = 32
DENSE_IN = 13
B = 4096
NF = F + 1
NPAIR = NF * (NF - 1) // 2

BF = B * F

try:
    _info = plsc.get_sparse_core_info()
    _NC, _NS = _info.num_cores, _info.num_subcores
except ValueError:
    _NC, _NS = 2, 16
NW = _NC * _NS
BPW = BF // NW
CH = 128


_RCB = 12800
_RGRID = -(-F * V // _RCB)
PACKED_ROWS = _RGRID * (_RCB // 4)


_RQ = _RCB // 4


def _repack_body(in_ref, out_ref):
    out_ref[:] = jnp.concatenate(
        [jnp.transpose(in_ref[:, u * _RQ:(u + 1) * _RQ], (1, 0))
         for u in range(4)], axis=1)


def _repack(tableT):
    return pl.pallas_call(
        _repack_body,
        grid=(_RGRID,),
        in_specs=[pl.BlockSpec((D, _RCB), lambda i: (0, i))],
        out_specs=pl.BlockSpec((_RCB // 4, 128), lambda i: (i, 0)),
        out_shape=jax.ShapeDtypeStruct((PACKED_ROWS, 128), jnp.float32),
    )(tableT)


def _sc_gather_body(packed_hbm, q_hbm, out_hbm, q_v, w0, w1, sem0, sem1):
    wid = lax.axis_index("s") * _NC + lax.axis_index("c")
    pltpu.sync_copy(q_hbm.at[:, wid, :, :], q_v)
    bufs = (w0, w1)
    sems = (sem0, sem1)
    cps = [None, None]
    for j in range(F):
        cps[j % 2] = pltpu.async_copy(
            packed_hbm.at[q_v.at[j // 8, j % 8]], bufs[j % 2], sems[j % 2])
        if j >= 1:
            k = (j - 1) % 2
            cps[k].wait()
            pltpu.sync_copy(
                bufs[k], out_hbm.at[pl.ds(wid * CH, CH), j - 1, :])
    k = (F - 1) % 2
    cps[k].wait()
    pltpu.sync_copy(bufs[k], out_hbm.at[pl.ds(wid * CH, CH), F - 1, :])


@functools.cache
def _sc_gather():
    return pl.kernel(
        _sc_gather_body,
        mesh=plsc.VectorSubcoreMesh(core_axis_name="c", subcore_axis_name="s"),
        out_type=jax.ShapeDtypeStruct((B, F, 128), jnp.float32),
        scratch_types=[
            pltpu.VMEM((4, 8, CH), jnp.int32),
            pltpu.VMEM((CH, 128), jnp.float32),
            pltpu.VMEM((CH, 128), jnp.float32),
            pltpu.SemaphoreType.DMA,
            pltpu.SemaphoreType.DMA,
        ],
        compiler_params=pltpu.CompilerParams(use_tc_tiling_on_sc=False),
    )


BB = 512
NBLK = B // BB


def _tc_body(dense_ref, wide_ref, sub_ref, dW0, db0, dW1, db1, dW2, db2,
             oW0a, Wt, ob0, oW1, ob1, oW2, ob2, oW3, ob3, out_ref):
    f32 = jnp.float32
    h = dense_ref[:]
    h = jnp.maximum(jnp.dot(h, dW0[:], preferred_element_type=f32) + db0[:], 0.0)
    h = jnp.maximum(jnp.dot(h, dW1[:], preferred_element_type=f32) + db1[:], 0.0)
    h = jnp.maximum(jnp.dot(h, dW2[:], preferred_element_type=f32) + db2[:], 0.0)
    parts = [h]
    for f in range(F):
        wf = wide_ref[:, f, :]
        sf = jnp.concatenate(
            [jnp.transpose(sub_ref[f // 8, g, f % 8, :].reshape(1, CH), (1, 0))
             for g in range(BB // CH)], axis=0)
        x = jnp.where(sf == 0, wf[:, 0:32],
            jnp.where(sf == 1, wf[:, 32:64],
            jnp.where(sf == 2, wf[:, 64:96], wf[:, 96:128])))
        parts.append(x)
    X = jnp.concatenate(parts, axis=1)
    r1 = lax.broadcasted_iota(jnp.int32, (D, NF * D), 1)
    r0 = lax.broadcasted_iota(jnp.int32, (D, NF * D), 0)
    E = (r1 % D == r0).astype(f32)
    s0 = lax.broadcasted_iota(jnp.int32, (NF * D, NF), 0)
    s1 = lax.broadcasted_iota(jnp.int32, (NF * D, NF), 1)
    ONES = (s0 // D == s1).astype(f32)
    acc = jnp.dot(h, oW0a[:], preferred_element_type=f32) + ob0[:]
    for n in range(NF):
        Xn = X[:, D * n:D * (n + 1)]
        Tn = jnp.dot(Xn, E, preferred_element_type=f32)
        Pn = X * Tn
        Gn = jnp.dot(Pn, ONES, preferred_element_type=f32)
        acc = acc + jnp.dot(Gn, Wt[n], preferred_element_type=f32)
    h = jnp.maximum(acc, 0.0)
    h = jnp.maximum(jnp.dot(h, oW1[:], preferred_element_type=f32) + ob1[:], 0.0)
    h = jnp.maximum(jnp.dot(h, oW2[:], preferred_element_type=f32) + ob2[:], 0.0)
    out_ref[:] = jnp.dot(h, oW3[:], preferred_element_type=f32) + ob3[:]


def _full(shape):
    nd = len(shape)
    return pl.BlockSpec(shape, lambda i, _nd=nd: (0,) * _nd)


def _tile_view(a):
    ap = jnp.pad(a, ((0, 0), (0, 32 - F)))
    return ap.reshape(32, 128, 4, 8).transpose(2, 0, 3, 1)


def kernel(dense_features, sparse_indices, table, dW0, db0, dW1, db1, dW2, db2,
           oW0, ob0, oW1, ob1, oW2, ob2, oW3, ob3):
    si = sparse_indices.astype(jnp.int32)
    offs = (jnp.arange(F, dtype=jnp.int32) * V)[None, :]
    flat = si + offs
    q4 = _tile_view((flat // _RCB) * _RQ + flat % _RQ)
    sub4 = _tile_view((flat % _RCB) // _RQ)

    packed = _repack(table.T)
    wide = _sc_gather()(packed, q4)

    ti0, ti1 = np.triu_indices(NF, 1)
    P = np.zeros((NF * NF, NPAIR), dtype=np.float32)
    P[ti0 * NF + ti1, np.arange(NPAIR)] = 1.0
    Wt = (jnp.asarray(P) @ oW0[D:]).reshape(NF, NF, oW0.shape[1])

    b2 = lambda x: x.reshape(1, -1)
    grid_spec = pl.GridSpec(
        grid=(NBLK,),
        in_specs=[
            pl.BlockSpec((BB, DENSE_IN), lambda i: (i, 0)),
            pl.BlockSpec((BB, F, 128), lambda i: (i, 0, 0)),
            pl.BlockSpec((4, BB // CH, 8, CH), lambda i: (0, i, 0, 0)),
            _full(dW0.shape), _full((1, 512)),
            _full(dW1.shape), _full((1, 256)),
            _full(dW2.shape), _full((1, D)),
            _full((D, 512)), _full(Wt.shape), _full((1, 512)),
            _full(oW1.shape), _full((1, 512)),
            _full(oW2.shape), _full((1, 256)),
            _full(oW3.shape), _full((1, 1)),
        ],
        out_specs=pl.BlockSpec((BB, 1), lambda i: (i, 0)),
    )
    logits = pl.pallas_call(
        _tc_body,
        grid_spec=grid_spec,
        out_shape=jax.ShapeDtypeStruct((B, 1), jnp.float32),
    )(dense_features, wide, sub4, dW0, b2(db0), dW1, b2(db1), dW2, b2(db2),
      oW0[:D], Wt, b2(ob0), oW1, b2(ob1), oW2, b2(ob2), oW3, b2(ob3))
    return logits
